# Initial kernel scaffold; baseline (speedup 1.0000x reference)
import jax
import jax.numpy as jnp
from jax.experimental import pallas as pl


def kernel(x, w1, b1, w2, b2):
    raise NotImplementedError("write your pallas kernel here")


# speedup vs baseline: 1.1021x; 1.1021x over previous
import functools

import jax
import jax.numpy as jnp
from jax.experimental import pallas as pl
from jax.experimental.pallas import tpu as pltpu

_LANE = 128
_SUBLANE = 8
_VMEM_LIMIT = 64 * 1024 * 1024
_BLOCK_TARGET_BYTES = 4 * 1024 * 1024


def _round_up(x, m):
    return (x + m - 1) // m * m


def _fused_body(w1t_ref, b1_ref, w2_ref, b2_ref, x_ref, o_ref, *, inv_s):
    xs = x_ref[...]
    pooled = jnp.sum(xs, axis=-1, dtype=jnp.float32) * inv_s

    w1t = w1t_ref[...]
    b1 = b1_ref[...]
    w2 = w2_ref[...]
    b2 = b2_ref[...]

    hid_a = jnp.sum(w1t[None, :, :] * pooled[:, :, None], axis=1) + b1
    hid_a = jnp.maximum(hid_a, 0.0)
    z = jnp.sum(w2[None, :, :] * hid_a[:, None, :], axis=-1) + b2[:, 0][None, :]
    gate = 1.0 / (1.0 + jnp.exp(-z))

    o_ref[...] = xs * gate.astype(xs.dtype)[:, :, None]


def kernel(x, w1, b1, w2, b2):
    b, c, d, hh, ww = x.shape
    s = d * hh * ww
    hidden = w1.shape[0]
    itemsize = jnp.dtype(x.dtype).itemsize
    inv_s = 1.0 / float(s)

    x_flat = x.reshape(b, c, s)
    w1t = jnp.asarray(w1, jnp.float32).T
    b1r = jnp.asarray(b1, jnp.float32).reshape(1, hidden)
    w2m = jnp.asarray(w2, jnp.float32)
    b2c = jnp.asarray(b2, jnp.float32).reshape(c, 1)

    slab_bytes = _round_up(c, _SUBLANE) * _round_up(s, _LANE) * itemsize
    bb = 1
    for cand in range(1, b + 1):
        if b % cand == 0 and cand * slab_bytes <= _BLOCK_TARGET_BYTES:
            bb = cand
    n_b = b // bb

    out_flat = pl.pallas_call(
        functools.partial(_fused_body, inv_s=inv_s),
        out_shape=jax.ShapeDtypeStruct((b, c, s), x.dtype),
        grid=(n_b,),
        in_specs=[
            pl.BlockSpec(w1t.shape, lambda i: (0, 0)),
            pl.BlockSpec(b1r.shape, lambda i: (0, 0)),
            pl.BlockSpec(w2m.shape, lambda i: (0, 0)),
            pl.BlockSpec(b2c.shape, lambda i: (0, 0)),
            pl.BlockSpec((bb, c, s), lambda i: (i, 0, 0)),
        ],
        out_specs=pl.BlockSpec((bb, c, s), lambda i: (i, 0, 0)),
        compiler_params=pltpu.CompilerParams(
            dimension_semantics=("parallel",),
            vmem_limit_bytes=_VMEM_LIMIT),
        cost_estimate=pl.CostEstimate(
            flops=2 * b * c * s, transcendentals=b * c,
            bytes_accessed=2 * b * c * s * itemsize),
    )(w1t, b1r, w2m, b2c, x_flat)

    return out_flat.reshape(b, c, d, hh, ww)


# speedup vs baseline: 1.1222x; 1.0183x over previous
import functools

import jax
import jax.numpy as jnp
from jax.experimental import pallas as pl
from jax.experimental.pallas import tpu as pltpu

_LANE = 128
_SUBLANE = 8
_VMEM_LIMIT = 64 * 1024 * 1024
_BLOCK_TARGET_BYTES = 8 * 1024 * 1024


def _round_up(x, m):
    return (x + m - 1) // m * m


def _fused_body(w1t_ref, b1_ref, w2_ref, b2_ref, x_ref, o_ref, *, inv_s):
    xs = x_ref[...]
    pooled = jnp.sum(xs, axis=-1, dtype=jnp.float32) * inv_s

    w1t = w1t_ref[...]
    b1 = b1_ref[...]
    w2 = w2_ref[...]
    b2 = b2_ref[...]

    hid_a = jnp.sum(w1t[None, :, :] * pooled[:, :, None], axis=1) + b1
    hid_a = jnp.maximum(hid_a, 0.0)
    z = jnp.sum(w2[None, :, :] * hid_a[:, None, :], axis=-1) + b2[:, 0][None, :]
    gate = 1.0 / (1.0 + jnp.exp(-z))

    o_ref[...] = xs * gate.astype(xs.dtype)[:, :, None]


def kernel(x, w1, b1, w2, b2):
    b, c, d, hh, ww = x.shape
    s = d * hh * ww
    hidden = w1.shape[0]
    itemsize = jnp.dtype(x.dtype).itemsize
    inv_s = 1.0 / float(s)

    x_flat = x.reshape(b, c, s)
    w1t = jnp.asarray(w1, jnp.float32).T
    b1r = jnp.asarray(b1, jnp.float32).reshape(1, hidden)
    w2m = jnp.asarray(w2, jnp.float32)
    b2c = jnp.asarray(b2, jnp.float32).reshape(c, 1)

    slab_bytes = _round_up(c, _SUBLANE) * _round_up(s, _LANE) * itemsize
    bb = 1
    for cand in range(1, b + 1):
        if b % cand == 0 and cand * slab_bytes <= _BLOCK_TARGET_BYTES:
            bb = cand
    n_b = b // bb

    out_flat = pl.pallas_call(
        functools.partial(_fused_body, inv_s=inv_s),
        out_shape=jax.ShapeDtypeStruct((b, c, s), x.dtype),
        grid=(n_b,),
        in_specs=[
            pl.BlockSpec(w1t.shape, lambda i: (0, 0)),
            pl.BlockSpec(b1r.shape, lambda i: (0, 0)),
            pl.BlockSpec(w2m.shape, lambda i: (0, 0)),
            pl.BlockSpec(b2c.shape, lambda i: (0, 0)),
            pl.BlockSpec((bb, c, s), lambda i: (i, 0, 0)),
        ],
        out_specs=pl.BlockSpec((bb, c, s), lambda i: (i, 0, 0)),
        compiler_params=pltpu.CompilerParams(
            dimension_semantics=("parallel",),
            vmem_limit_bytes=_VMEM_LIMIT),
        cost_estimate=pl.CostEstimate(
            flops=2 * b * c * s, transcendentals=b * c,
            bytes_accessed=2 * b * c * s * itemsize),
    )(w1t, b1r, w2m, b2c, x_flat)

    return out_flat.reshape(b, c, d, hh, ww)


# speedup vs baseline: 1.1495x; 1.0243x over previous
import functools

import jax
import jax.numpy as jnp
from jax.experimental import pallas as pl
from jax.experimental.pallas import tpu as pltpu

_NBUF = 8
_VMEM_LIMIT = 40 * 1024 * 1024


def _ring_body(w1t_ref, b1_ref, w2_ref, b2_ref, x_ref, o_ref,
               xbuf, obuf, in_sem, out_sem, *, n, cb, inv_s):
    nbuf = _NBUF

    def start_in(chunk, slot):
        pltpu.make_async_copy(
            x_ref.at[pl.ds(chunk * cb, cb)], xbuf.at[slot],
            in_sem.at[slot]).start()

    def wait_in(slot):
        pltpu.make_async_copy(
            x_ref.at[pl.ds(0, cb)], xbuf.at[slot], in_sem.at[slot]).wait()

    def start_out(chunk, slot):
        pltpu.make_async_copy(
            obuf.at[slot], o_ref.at[pl.ds(chunk * cb, cb)],
            out_sem.at[slot]).start()

    def wait_out(slot):
        pltpu.make_async_copy(
            obuf.at[slot], o_ref.at[pl.ds(0, cb)], out_sem.at[slot]).wait()

    for k in range(min(nbuf - 1, n)):
        start_in(k, k)

    def body(i, _):
        slot = jax.lax.rem(i, nbuf)

        @pl.when(i + nbuf - 1 < n)
        def _():
            start_in(i + nbuf - 1, jax.lax.rem(i + nbuf - 1, nbuf))

        wait_in(slot)

        @pl.when(i >= nbuf)
        def _():
            wait_out(slot)

        xs = xbuf[slot]
        pooled = jnp.sum(xs, axis=-1, dtype=jnp.float32) * inv_s
        w1t = w1t_ref[...]
        hid = jnp.sum(w1t[None, :, :] * pooled[:, :, None], axis=1) \
            + b1_ref[...]
        hid = jnp.maximum(hid, 0.0)
        z = jnp.sum(w2_ref[...][None, :, :] * hid[:, None, :], axis=-1) \
            + b2_ref[...][:, 0][None, :]
        gate = 1.0 / (1.0 + jnp.exp(-z))
        obuf[slot] = xs * gate.astype(xs.dtype)[:, :, None]

        start_out(i, slot)
        return 0

    jax.lax.fori_loop(0, n, body, 0)

    for k in range(max(n - nbuf, 0), n):
        wait_out(k % nbuf)


def kernel(x, w1, b1, w2, b2):
    b, c, d, hh, ww = x.shape
    s = d * hh * ww
    hidden = w1.shape[0]
    inv_s = 1.0 / float(s)

    x_flat = x.reshape(b, c, s)
    w1t = jnp.asarray(w1, jnp.float32).T
    b1r = jnp.asarray(b1, jnp.float32).reshape(1, hidden)
    w2m = jnp.asarray(w2, jnp.float32)
    b2c = jnp.asarray(b2, jnp.float32).reshape(c, 1)

    cb = 1
    n = b // cb

    out_flat = pl.pallas_call(
        functools.partial(_ring_body, n=n, cb=cb, inv_s=inv_s),
        out_shape=jax.ShapeDtypeStruct((b, c, s), x.dtype),
        in_specs=[
            pl.BlockSpec(memory_space=pltpu.MemorySpace.VMEM),
            pl.BlockSpec(memory_space=pltpu.MemorySpace.VMEM),
            pl.BlockSpec(memory_space=pltpu.MemorySpace.VMEM),
            pl.BlockSpec(memory_space=pltpu.MemorySpace.VMEM),
            pl.BlockSpec(memory_space=pl.ANY),
        ],
        out_specs=pl.BlockSpec(memory_space=pl.ANY),
        scratch_shapes=[
            pltpu.VMEM((_NBUF, cb, c, s), x.dtype),
            pltpu.VMEM((_NBUF, cb, c, s), x.dtype),
            pltpu.SemaphoreType.DMA((_NBUF,)),
            pltpu.SemaphoreType.DMA((_NBUF,)),
        ],
        compiler_params=pltpu.CompilerParams(
            vmem_limit_bytes=_VMEM_LIMIT),
        cost_estimate=pl.CostEstimate(
            flops=2 * b * c * s, transcendentals=b * c,
            bytes_accessed=2 * b * c * s * 4),
    )(w1t, b1r, w2m, b2c, x_flat)

    return out_flat.reshape(b, c, d, hh, ww)
